# baseline (speedup 1.0000x reference)
import functools

import jax
import jax.numpy as jnp
from jax import lax
from jax.experimental import pallas as pl
from jax.experimental.pallas import tpu as pltpu
from jax.experimental.pallas import tpu_sc as plsc

_VOCAB = 1_000_000
_D = 64
_B = 16384
_W = 128
_NC = 2
_NS = 16
_L = 16
_NW = _NC * _NS
_BPW = _B // _NW
_HALF = _BPW // 2
_NGRP = _HALF // _L

_mesh = plsc.VectorSubcoreMesh(
    core_axis_name="c", subcore_axis_name="s", num_cores=_NC, num_subcores=_NS
)


@functools.partial(
    pl.kernel,
    out_type=jax.ShapeDtypeStruct((_B,), jnp.float32),
    mesh=_mesh,
    compiler_params=pltpu.CompilerParams(needs_layout_passes=False),
    scratch_types=[
        pltpu.VMEM((_BPW,), jnp.int32),
        pltpu.VMEM((_BPW,), jnp.int32),
        pltpu.VMEM((_HALF, _W), jnp.float32),
        pltpu.VMEM((_HALF, _W), jnp.float32),
        pltpu.VMEM((_BPW,), jnp.float32),
        pltpu.SemaphoreType.DMA,
        pltpu.SemaphoreType.DMA,
    ],
)
def _glove_sc(left_hbm, right_hbm, laug_hbm, raug_hbm,
              out_hbm, lids, rids, lrow, rrow, outv, sem0, sem1):
    wid = lax.axis_index("s") * _NC + lax.axis_index("c")
    base = wid * _BPW

    pltpu.sync_copy(left_hbm.at[pl.ds(base, _BPW)], lids)
    pltpu.sync_copy(right_hbm.at[pl.ds(base, _BPW)], rids)

    lane = lax.iota(jnp.int32, _L)
    bias_col = jnp.full((_L,), _D, jnp.int32)

    for h in range(2):
        hb = h * _HALF
        c0 = pltpu.async_copy(laug_hbm.at[lids.at[pl.ds(hb, _HALF)]], lrow, sem0)
        c1 = pltpu.async_copy(raug_hbm.at[rids.at[pl.ds(hb, _HALF)]], rrow, sem1)
        c0.wait()
        c1.wait()

        def group(g, carry):
            rows = jnp.full((_L,), g * _L, jnp.int32) + lane
            acc = plsc.load_gather(lrow, [rows, bias_col]) + plsc.load_gather(
                rrow, [rows, bias_col])
            for c in range(_D):
                col = jnp.full((_L,), c, jnp.int32)
                acc = acc + plsc.load_gather(lrow, [rows, col]) * plsc.load_gather(
                    rrow, [rows, col])
            outv[pl.ds(pl.multiple_of(hb + g * _L, _L), _L)] = acc
            return carry

        lax.fori_loop(0, _NGRP, group, 0)

    pltpu.sync_copy(outv, out_hbm.at[pl.ds(base, _BPW)])


def kernel(left_id, right_id, l_emb, l_bias, r_emb, r_bias):
    pad = jnp.zeros((_VOCAB, _W - _D - 1), jnp.float32)
    laug = jnp.concatenate([l_emb, l_bias, pad], axis=1)
    raug = jnp.concatenate([r_emb, r_bias, pad], axis=1)
    return _glove_sc(
        left_id.astype(jnp.int32), right_id.astype(jnp.int32), laug, raug
    )


# speedup vs baseline: 1.4884x; 1.4884x over previous
import functools

import jax
import jax.numpy as jnp
from jax import lax
from jax.experimental import pallas as pl
from jax.experimental.pallas import tpu as pltpu
from jax.experimental.pallas import tpu_sc as plsc

_VOCAB = 1_000_000
_D = 64
_B = 16384
_W = 128
_BROWS = _VOCAB // 2
_BBIAS = (_VOCAB + _W - 1) // _W
_NC = 2
_NS = 16
_L = 16
_NW = _NC * _NS
_BPW = _B // _NW
_Q = 128
_NQ = _BPW // _Q
_NGRP = _Q // _L

_mesh = plsc.VectorSubcoreMesh(
    core_axis_name="c", subcore_axis_name="s", num_cores=_NC, num_subcores=_NS
)


@functools.partial(
    pl.kernel,
    out_type=jax.ShapeDtypeStruct((_B,), jnp.float32),
    mesh=_mesh,
    compiler_params=pltpu.CompilerParams(needs_layout_passes=False),
    scratch_types=[
        pltpu.VMEM((_BPW,), jnp.int32),
        pltpu.VMEM((_BPW,), jnp.int32),
        pltpu.VMEM((_BPW,), jnp.int32),
        pltpu.VMEM((_BPW,), jnp.int32),
        pltpu.VMEM((_BPW,), jnp.int32),
        pltpu.VMEM((_BPW,), jnp.int32),
        pltpu.VMEM((_Q, _W), jnp.float32),
        pltpu.VMEM((_Q, _W), jnp.float32),
        pltpu.VMEM((_Q, _W), jnp.float32),
        pltpu.VMEM((_Q, _W), jnp.float32),
        pltpu.VMEM((_BPW,), jnp.float32),
        pltpu.SemaphoreType.DMA,
        pltpu.SemaphoreType.DMA,
        pltpu.SemaphoreType.DMA,
        pltpu.SemaphoreType.DMA,
    ],
)
def _glove_sc(left_hbm, right_hbm, ltab_hbm, lbias_hbm, rtab_hbm, rbias_hbm,
              out_hbm, lids, rids, ltix, rtix, lbix, rbix,
              lrow, rrow, lbrow, rbrow, outv, sem0, sem1, sem2, sem3):
    wid = lax.axis_index("s") * _NC + lax.axis_index("c")
    base = wid * _BPW

    pltpu.sync_copy(left_hbm.at[pl.ds(base, _BPW)], lids)
    pltpu.sync_copy(right_hbm.at[pl.ds(base, _BPW)], rids)

    def derive(g, carry):
        gb = pl.multiple_of(g * _L, _L)
        vl = lids[pl.ds(gb, _L)]
        vr = rids[pl.ds(gb, _L)]
        ltix[pl.ds(gb, _L)] = vl >> 1
        rtix[pl.ds(gb, _L)] = vr >> 1
        lbix[pl.ds(gb, _L)] = vl >> 7
        rbix[pl.ds(gb, _L)] = vr >> 7
        return carry

    lax.fori_loop(0, _BPW // _L, derive, 0)

    lane = lax.iota(jnp.int32, _L)

    for q in range(_NQ):
        qb = q * _Q
        c0 = pltpu.async_copy(ltab_hbm.at[ltix.at[pl.ds(qb, _Q)]], lrow, sem0)
        c1 = pltpu.async_copy(rtab_hbm.at[rtix.at[pl.ds(qb, _Q)]], rrow, sem1)
        c2 = pltpu.async_copy(lbias_hbm.at[lbix.at[pl.ds(qb, _Q)]], lbrow, sem2)
        c3 = pltpu.async_copy(rbias_hbm.at[rbix.at[pl.ds(qb, _Q)]], rbrow, sem3)
        c0.wait()
        c1.wait()
        c2.wait()
        c3.wait()

        def group(g, carry):
            gb = pl.multiple_of(g * _L, _L)
            rows = jnp.full((_L,), g * _L, jnp.int32) + lane
            vl = lids[pl.ds(pl.multiple_of(qb + gb, _L), _L)]
            vr = rids[pl.ds(pl.multiple_of(qb + gb, _L), _L)]
            lhalf = (vl & 1) * _D
            rhalf = (vr & 1) * _D
            acc = plsc.load_gather(lbrow, [rows, vl & (_W - 1)]) + plsc.load_gather(
                rbrow, [rows, vr & (_W - 1)])
            for c in range(_D):
                acc = acc + plsc.load_gather(lrow, [rows, lhalf + c]) * plsc.load_gather(
                    rrow, [rows, rhalf + c])
            outv[pl.ds(pl.multiple_of(qb + gb, _L), _L)] = acc
            return carry

        lax.fori_loop(0, _NGRP, group, 0)

    pltpu.sync_copy(outv, out_hbm.at[pl.ds(base, _BPW)])


def kernel(left_id, right_id, l_emb, l_bias, r_emb, r_bias):
    ltab = l_emb.reshape(_BROWS, _W)
    rtab = r_emb.reshape(_BROWS, _W)
    pad = _BBIAS * _W - _VOCAB
    lbias2 = jnp.pad(l_bias.reshape(_VOCAB), (0, pad)).reshape(_BBIAS, _W)
    rbias2 = jnp.pad(r_bias.reshape(_VOCAB), (0, pad)).reshape(_BBIAS, _W)
    return _glove_sc(
        left_id.astype(jnp.int32), right_id.astype(jnp.int32),
        ltab, lbias2, rtab, rbias2,
    )
